# Initial kernel scaffold; baseline (speedup 1.0000x reference)
import jax
import jax.numpy as jnp
from jax.experimental import pallas as pl


def kernel(atom_feat, rdf_feat, bdf_feat, atom_idx, ele_idx, graph_idx, ref_feat, params):
    raise NotImplementedError("write your pallas kernel here")


# speedup vs baseline: 1.5271x; 1.5271x over previous
import jax
import jax.numpy as jnp
from jax import lax
from jax.experimental import pallas as pl
from jax.experimental.pallas import tpu as pltpu
from jax.experimental.pallas import tpu_sc as plsc

F32 = jnp.float32
N = 320000
AE = 128
N_ATOMS = 10000
N_ELE = 100
N_ELE_P = 104
N_GRAPHS = 1024
NC, NS = 2, 16
NW = NC * NS
CH = 80
NCHUNKS = N // CH
BR = 1280
GRID = N // BR


def _relu(x):
    return jnp.maximum(x, 0.0)


def _dot(a, b):
    return jnp.dot(a, b, preferred_element_type=F32)


def _row_spec(b, w):
    return pl.BlockSpec((b, w), lambda i: (i, 0))


def _full(shape):
    return pl.BlockSpec(shape, lambda i: (0,) * len(shape))


def _t1_body(af, rdf, bdf, wa, ba, wr, br_, wb, bb, h0a, h0b, xr, xb):
    a = af[...]
    h0a[...] = _relu(_dot(a[:, :AE], wa[...]) + ba[...])
    h0b[...] = _relu(_dot(a[:, AE:], wa[...]) + ba[...])
    xr[...] = jnp.clip(_dot(rdf[...], wr[...]) + br_[...], 0.0, 6.0)
    xb[...] = jnp.clip(_dot(bdf[...], wb[...]) + bb[...], 0.0, 6.0)


def _t1(af, rdf, bdf, wa, ba, wr, br_, wb, bb):
    return pl.pallas_call(
        _t1_body,
        grid=(GRID,),
        in_specs=[_row_spec(BR, 256), _row_spec(BR, 128), _row_spec(BR, 128),
                  _full((128, 128)), _full((1, 128)),
                  _full((128, 128)), _full((1, 128)),
                  _full((128, 128)), _full((1, 128))],
        out_specs=[_row_spec(BR, 128)] * 4,
        out_shape=[jax.ShapeDtypeStruct((N, 128), F32)] * 4,
    )(af, rdf, bdf, wa, ba, wr, br_, wb, bb)


def _t3_body(xr, xb, pa, pe, wdr, war, wer, br_, wdb, wab, web, bb,
             u_ref, v_ref, st_ref):
    i = pl.program_id(0)
    u = (_dot(xr[...], wdr[...]) + _dot(pa[...], war[...])
         + _dot(pe[...], wer[...]) + br_[...])
    v = (_dot(xb[...], wdb[...]) + _dot(pa[...], wab[...])
         + _dot(pe[...], web[...]) + bb[...])
    u_ref[...] = u
    v_ref[...] = v
    su = jnp.sum(u, axis=0, keepdims=True)
    squ = jnp.sum(u * u, axis=0, keepdims=True)
    sv = jnp.sum(v, axis=0, keepdims=True)
    sqv = jnp.sum(v * v, axis=0, keepdims=True)
    rows = jnp.concatenate([su, squ, sv, sqv, jnp.zeros((4, 2 * AE), F32)], axis=0)

    @pl.when(i == 0)
    def _():
        st_ref[...] = jnp.zeros_like(st_ref)

    st_ref[...] += rows


def _t3(xr, xb, pa, pe, wdr, war, wer, br_, wdb, wab, web, bb):
    return pl.pallas_call(
        _t3_body,
        grid=(GRID,),
        in_specs=[_row_spec(BR, 128)] * 4
        + [_full((128, 256)), _full((128, 256)), _full((128, 256)), _full((1, 256)),
           _full((128, 256)), _full((128, 256)), _full((128, 256)), _full((1, 256))],
        out_specs=[_row_spec(BR, 256), _row_spec(BR, 256), _full((8, 256))],
        out_shape=[jax.ShapeDtypeStruct((N, 256), F32),
                   jax.ShapeDtypeStruct((N, 256), F32),
                   jax.ShapeDtypeStruct((8, 256), F32)],
    )(xr, xb, pa, pe, wdr, war, wer, br_, wdb, wab, web, bb)


def _t4_body(u, v, h0a, h0b, st, gr, btr, gb, btb, wm1, wm2, bm, h1a, h1b):
    st_ = st[...]
    inv_n = 1.0 / N
    mu = st_[0:1] * inv_n
    vu = st_[1:2] * inv_n - mu * mu
    su_ = gr[...] * lax.rsqrt(vu + 1e-5)
    shu = btr[...] - mu * su_
    mv = st_[2:3] * inv_n
    vv = st_[3:4] * inv_n - mv * mv
    sv_ = gb[...] * lax.rsqrt(vv + 1e-5)
    shv = btb[...] - mv * sv_
    h0 = jnp.concatenate([h0a[...], h0b[...]], axis=1)
    x1 = _relu(u[...] * su_ + shu + h0)
    x2 = _relu(v[...] * sv_ + shv + h0)
    h1 = _relu(_dot(x1, wm1[...]) + _dot(x2, wm2[...]) + bm[...])
    h1a[...] = h1[:, :AE]
    h1b[...] = h1[:, AE:]


def _t4(u, v, h0a, h0b, st, gr, btr, gb, btb, wm1, wm2, bm):
    return pl.pallas_call(
        _t4_body,
        grid=(GRID,),
        in_specs=[_row_spec(BR, 256), _row_spec(BR, 256),
                  _row_spec(BR, 128), _row_spec(BR, 128),
                  _full((8, 256)),
                  _full((1, 256)), _full((1, 256)), _full((1, 256)), _full((1, 256)),
                  _full((256, 256)), _full((256, 256)), _full((1, 256))],
        out_specs=[_row_spec(BR, 128)] * 2,
        out_shape=[jax.ShapeDtypeStruct((N, 128), F32)] * 2,
    )(u, v, h0a, h0b, st, gr, btr, gb, btb, wm1, wm2, bm)


def _fin_body(part, cnt, out):
    s = part[0] + part[1]
    c = cnt[0, :, 0:1] + cnt[1, :, 0:1]
    out[...] = _relu(s / jnp.maximum(c, 1.0))


def _fin(k, part, cnt):
    return pl.pallas_call(
        _fin_body,
        grid=(1,),
        in_specs=[_full((NC, k, 128)), _full((NC, k, 128))],
        out_specs=_full((k, 128)),
        out_shape=jax.ShapeDtypeStruct((k, 128), F32),
    )(part, cnt)


def _t8_body(gpa, gpb, gcnt, ref128, w1a, w1b, w1r, b1, w2, b2, out):
    c = jnp.maximum(gcnt[0, :, 0:1] + gcnt[1, :, 0:1], 1.0)
    ga = (gpa[0] + gpa[1]) / c
    gb_ = (gpb[0] + gpb[1]) / c
    t = _relu(_dot(ga, w1a[...]) + _dot(gb_, w1b[...])
              + ref128[...] * w1r[...] + b1[...])
    out[...] = _dot(t, w2[...]) + b2[...]


def _t8(gpa, gpb, gcnt, ref128, w1a, w1b, w1r, b1, w2, b2):
    return pl.pallas_call(
        _t8_body,
        grid=(1,),
        in_specs=[_full((NC, N_GRAPHS, 128)), _full((NC, N_GRAPHS, 128)),
                  _full((NC, N_GRAPHS, 128)), _full((N_GRAPHS, 128)),
                  _full((128, 128)), _full((128, 128)), _full((1, 128)),
                  _full((1, 128)), _full((128, 128)), _full((1, 128))],
        out_specs=_full((N_GRAPHS, 128)),
        out_shape=jax.ShapeDtypeStruct((N_GRAPHS, 128), F32),
    )(gpa, gpb, gcnt, ref128, w1a, w1b, w1r, b1, w2, b2)


def _ranges(k):
    rp = (k // (NS * 8)) * 8
    rem = k - rp * NS
    return rp, rem


def _chunks(total):
    off = 0
    while off < total:
        sz = min(CH, total - off)
        yield off, sz
        off += sz


def _zero_rows(dst, zbuf, k, sid):
    rp, rem = _ranges(k)
    if rp:
        for off, sz in _chunks(rp):
            pltpu.sync_copy(zbuf.at[pl.ds(0, sz)], dst.at[pl.ds(sid * rp + off, sz)])
    if rem:
        @pl.when(sid == 0)
        def _():
            for off, sz in _chunks(rem):
                pltpu.sync_copy(zbuf.at[pl.ds(0, sz)], dst.at[pl.ds(rp * NS + off, sz)])


def _writeout(dst, src, stage, k, cid, sid):
    rp, rem = _ranges(k)
    if rp:
        for off, sz in _chunks(rp):
            pltpu.sync_copy(src.at[pl.ds(sid * rp + off, sz)], stage.at[pl.ds(0, sz)])
            pltpu.sync_copy(stage.at[pl.ds(0, sz)],
                            dst.at[cid, pl.ds(sid * rp + off, sz)])
    if rem:
        @pl.when(sid == 0)
        def _():
            for off, sz in _chunks(rem):
                pltpu.sync_copy(src.at[pl.ds(rp * NS + off, sz)], stage.at[pl.ds(0, sz)])
                pltpu.sync_copy(stage.at[pl.ds(0, sz)],
                                dst.at[cid, pl.ds(rp * NS + off, sz)])


def _make_segsum(k):
    mesh = plsc.VectorSubcoreMesh(core_axis_name="c", subcore_axis_name="s")

    def body(xh, ih, zh, oh, iv, bv, ts):
        cid = lax.axis_index("c")
        sid = lax.axis_index("s")
        wid = sid * NC + cid
        pltpu.sync_copy(zh, bv)
        _zero_rows(ts, bv, k, sid)
        plsc.subcore_barrier()

        def step(i, carry):
            r = (i * NW + wid) * CH
            pltpu.sync_copy(ih.at[pl.ds(r, CH)], iv)
            pltpu.sync_copy(xh.at[pl.ds(r, CH)], bv)
            pltpu.sync_copy(bv, ts.at[iv], add=True)
            return carry

        lax.fori_loop(0, NCHUNKS // NW, step, 0)
        plsc.subcore_barrier()
        _writeout(oh, ts, bv, k, cid, sid)

    return pl.kernel(
        body, mesh=mesh,
        out_type=[jax.ShapeDtypeStruct((NC, k, 128), F32)],
        scratch_types=[pltpu.VMEM((CH,), jnp.int32), pltpu.VMEM((CH, 128), F32),
                       pltpu.VMEM_SHARED((k, 128), F32)])


def _make_counts(k):
    mesh = plsc.VectorSubcoreMesh(core_axis_name="c", subcore_axis_name="s")

    def body(ih, zh, onesh, oh, iv, bv, ts):
        cid = lax.axis_index("c")
        sid = lax.axis_index("s")
        wid = sid * NC + cid
        pltpu.sync_copy(zh, bv)
        _zero_rows(ts, bv, k, sid)
        plsc.subcore_barrier()
        pltpu.sync_copy(onesh, bv)

        def step(i, carry):
            r = (i * NW + wid) * CH
            pltpu.sync_copy(ih.at[pl.ds(r, CH)], iv)
            pltpu.sync_copy(bv, ts.at[iv], add=True)
            return carry

        lax.fori_loop(0, NCHUNKS // NW, step, 0)
        plsc.subcore_barrier()
        _writeout(oh, ts, bv, k, cid, sid)

    return pl.kernel(
        body, mesh=mesh,
        out_type=[jax.ShapeDtypeStruct((NC, k, 128), F32)],
        scratch_types=[pltpu.VMEM((CH,), jnp.int32), pltpu.VMEM((CH, 128), F32),
                       pltpu.VMEM_SHARED((k, 128), F32)])


def _make_gather():
    mesh = plsc.VectorSubcoreMesh(core_axis_name="c", subcore_axis_name="s")
    out_type = [jax.ShapeDtypeStruct((N, 128), F32),
                jax.ShapeDtypeStruct((N, 128), F32)]
    scr = [pltpu.VMEM((CH,), jnp.int32), pltpu.VMEM((CH,), jnp.int32),
           pltpu.VMEM((CH, 128), F32), pltpu.VMEM((CH, 128), F32),
           pltpu.SemaphoreType.DMA, pltpu.SemaphoreType.DMA]

    def body(t1h, t2h, i1h, i2h, g1h, g2h, i1v, i2v, b1v, b2v, s1, s2):
        cid = lax.axis_index("c")
        sid = lax.axis_index("s")
        wid = sid * NC + cid

        def step(i, carry):
            r = (i * NW + wid) * CH
            pltpu.sync_copy(i1h.at[pl.ds(r, CH)], i1v)
            pltpu.sync_copy(i2h.at[pl.ds(r, CH)], i2v)
            d1 = pltpu.async_copy(t1h.at[i1v], b1v, s1)
            d2 = pltpu.async_copy(t2h.at[i2v], b2v, s2)
            d1.wait()
            d2.wait()
            pltpu.sync_copy(b1v, g1h.at[pl.ds(r, CH)])
            pltpu.sync_copy(b2v, g2h.at[pl.ds(r, CH)])
            return carry

        lax.fori_loop(0, NCHUNKS // NW, step, 0)

    return pl.kernel(body, mesh=mesh, out_type=out_type, scratch_types=scr)


def kernel(atom_feat, rdf_feat, bdf_feat, atom_idx, ele_idx, graph_idx,
           ref_feat, params):
    aidx = atom_idx.astype(jnp.int32)
    eidx = ele_idx.astype(jnp.int32)
    gidx = graph_idx.astype(jnp.int32)
    p = params

    wa = p["embed_atom"]["W"].T
    ba = p["embed_atom"]["b"][None]
    wr = p["embed_rdf"]["W"].T
    brr = p["embed_rdf"]["b"][None]
    wb = p["embed_bdf"]["W"].T
    bbb = p["embed_bdf"]["b"][None]
    h0a, h0b, xr, xb = _t1(atom_feat, rdf_feat, bdf_feat,
                           wa, ba, wr, brr, wb, bbb)

    za = jnp.zeros((CH, 128), F32)
    ones = jnp.ones((CH, 128), F32)
    seg_a = _make_segsum(N_ATOMS)
    seg_e = _make_segsum(N_ELE_P)
    acnt = _make_counts(N_ATOMS)(aidx, za, ones)[0]
    ecnt = _make_counts(N_ELE_P)(eidx, za, ones)[0]

    xa, xbh = h0a, h0b
    for li, mod in enumerate((p["dl1"], p["dl2"])):
        (ap,) = seg_a(xa, aidx, za)
        (ep,) = seg_e(xbh, eidx, za)
        A = _fin(N_ATOMS, ap, acnt)
        E = _fin(N_ELE_P, ep, ecnt)
        PA, PE = jnp.take(A, aidx, axis=0), jnp.take(E, eidx, axis=0)

        wtr = mod["rdf"]["fc1"]["W"].T
        wtb = mod["bdf"]["fc1"]["W"].T
        u, v, st = _t3(xr, xb, PA, PE,
                       wtr[:128], wtr[128:256], wtr[256:],
                       mod["rdf"]["fc1"]["b"][None],
                       wtb[:128], wtb[128:256], wtb[256:],
                       mod["bdf"]["fc1"]["b"][None])
        wm = mod["fc1"]["W"].T
        xa, xbh = _t4(u, v, xa, xbh, st,
                      mod["rdf"]["bn_gamma"][None], mod["rdf"]["bn_beta"][None],
                      mod["bdf"]["bn_gamma"][None], mod["bdf"]["bn_beta"][None],
                      wm[:256], wm[256:], mod["fc1"]["b"][None])

    (gpa,) = _make_segsum(N_GRAPHS)(xa, gidx, za)
    (gpb,) = _make_segsum(N_GRAPHS)(xbh, gidx, za)
    gcnt = _make_counts(N_GRAPHS)(gidx, za, ones)[0]

    w1t = p["fc1"]["W"].T
    w1a = jnp.pad(w1t[:128], ((0, 0), (0, 112)))
    w1b = jnp.pad(w1t[128:256], ((0, 0), (0, 112)))
    w1r = jnp.pad(w1t[256:257], ((0, 0), (0, 112)))
    b1 = jnp.pad(p["fc1"]["b"][None], ((0, 0), (0, 112)))
    w2 = jnp.pad(p["fc2"]["W"].T, ((0, 112), (0, 127)))
    b2 = jnp.pad(p["fc2"]["b"][None], ((0, 0), (0, 127)))
    ref128 = jnp.broadcast_to(ref_feat, (N_GRAPHS, 128))
    out = _t8(gpa, gpb, gcnt, ref128, w1a, w1b, w1r, b1, w2, b2)
    return out[:, :1]


# speedup vs baseline: 1.7830x; 1.1676x over previous
import jax
import jax.numpy as jnp
from jax import lax
from jax.experimental import pallas as pl
from jax.experimental.pallas import tpu as pltpu
from jax.experimental.pallas import tpu_sc as plsc

F32 = jnp.float32
N = 320000
AE = 128
N_ATOMS = 10000
N_ELE = 100
N_ELE_P = 104
N_GRAPHS = 1024
NC, NS = 2, 16
NW = NC * NS
CH = 80
NCHUNKS = N // CH
BR = 1280
GRID = N // BR


def _relu(x):
    return jnp.maximum(x, 0.0)


def _dot(a, b):
    return jnp.dot(a, b, preferred_element_type=F32)


def _row_spec(b, w):
    return pl.BlockSpec((b, w), lambda i: (i, 0))


def _full(shape):
    return pl.BlockSpec(shape, lambda i: (0,) * len(shape))


def _t1_body(af, rdf, bdf, wa, ba, wr, br_, wb, bb, h0a, h0b, xr, xb):
    a = af[...]
    h0a[...] = _relu(_dot(a[:, :AE], wa[...]) + ba[...])
    h0b[...] = _relu(_dot(a[:, AE:], wa[...]) + ba[...])
    xr[...] = jnp.clip(_dot(rdf[...], wr[...]) + br_[...], 0.0, 6.0)
    xb[...] = jnp.clip(_dot(bdf[...], wb[...]) + bb[...], 0.0, 6.0)


def _t1(af, rdf, bdf, wa, ba, wr, br_, wb, bb):
    return pl.pallas_call(
        _t1_body,
        grid=(GRID,),
        in_specs=[_row_spec(BR, 256), _row_spec(BR, 128), _row_spec(BR, 128),
                  _full((128, 128)), _full((1, 128)),
                  _full((128, 128)), _full((1, 128)),
                  _full((128, 128)), _full((1, 128))],
        out_specs=[_row_spec(BR, 128)] * 4,
        out_shape=[jax.ShapeDtypeStruct((N, 128), F32)] * 4,
    )(af, rdf, bdf, wa, ba, wr, br_, wb, bb)


def _t3_body(xr, xb, pa, pe, wdr, war, wer, br_, wdb, wab, web, bb,
             u_ref, v_ref, st_ref):
    i = pl.program_id(0)
    u = (_dot(xr[...], wdr[...]) + _dot(pa[...], war[...])
         + _dot(pe[...], wer[...]) + br_[...])
    v = (_dot(xb[...], wdb[...]) + _dot(pa[...], wab[...])
         + _dot(pe[...], web[...]) + bb[...])
    u_ref[...] = u
    v_ref[...] = v
    su = jnp.sum(u, axis=0, keepdims=True)
    squ = jnp.sum(u * u, axis=0, keepdims=True)
    sv = jnp.sum(v, axis=0, keepdims=True)
    sqv = jnp.sum(v * v, axis=0, keepdims=True)
    rows = jnp.concatenate([su, squ, sv, sqv, jnp.zeros((4, 2 * AE), F32)], axis=0)

    @pl.when(i == 0)
    def _():
        st_ref[...] = jnp.zeros_like(st_ref)

    st_ref[...] += rows


def _t3(xr, xb, pa, pe, wdr, war, wer, br_, wdb, wab, web, bb):
    return pl.pallas_call(
        _t3_body,
        grid=(GRID,),
        in_specs=[_row_spec(BR, 128)] * 4
        + [_full((128, 256)), _full((128, 256)), _full((128, 256)), _full((1, 256)),
           _full((128, 256)), _full((128, 256)), _full((128, 256)), _full((1, 256))],
        out_specs=[_row_spec(BR, 256), _row_spec(BR, 256), _full((8, 256))],
        out_shape=[jax.ShapeDtypeStruct((N, 256), F32),
                   jax.ShapeDtypeStruct((N, 256), F32),
                   jax.ShapeDtypeStruct((8, 256), F32)],
    )(xr, xb, pa, pe, wdr, war, wer, br_, wdb, wab, web, bb)


def _t4_body(u, v, h0a, h0b, st, gr, btr, gb, btb, wm1, wm2, bm, h1a, h1b):
    st_ = st[...]
    inv_n = 1.0 / N
    mu = st_[0:1] * inv_n
    vu = st_[1:2] * inv_n - mu * mu
    su_ = gr[...] * lax.rsqrt(vu + 1e-5)
    shu = btr[...] - mu * su_
    mv = st_[2:3] * inv_n
    vv = st_[3:4] * inv_n - mv * mv
    sv_ = gb[...] * lax.rsqrt(vv + 1e-5)
    shv = btb[...] - mv * sv_
    h0 = jnp.concatenate([h0a[...], h0b[...]], axis=1)
    x1 = _relu(u[...] * su_ + shu + h0)
    x2 = _relu(v[...] * sv_ + shv + h0)
    h1 = _relu(_dot(x1, wm1[...]) + _dot(x2, wm2[...]) + bm[...])
    h1a[...] = h1[:, :AE]
    h1b[...] = h1[:, AE:]


def _t4(u, v, h0a, h0b, st, gr, btr, gb, btb, wm1, wm2, bm):
    return pl.pallas_call(
        _t4_body,
        grid=(GRID,),
        in_specs=[_row_spec(BR, 256), _row_spec(BR, 256),
                  _row_spec(BR, 128), _row_spec(BR, 128),
                  _full((8, 256)),
                  _full((1, 256)), _full((1, 256)), _full((1, 256)), _full((1, 256)),
                  _full((256, 256)), _full((256, 256)), _full((1, 256))],
        out_specs=[_row_spec(BR, 128)] * 2,
        out_shape=[jax.ShapeDtypeStruct((N, 128), F32)] * 2,
    )(u, v, h0a, h0b, st, gr, btr, gb, btb, wm1, wm2, bm)


def _fin_body(part, cnt, out):
    s = part[0] + part[1]
    c = cnt[0, :, 0:1] + cnt[1, :, 0:1]
    out[...] = _relu(s / jnp.maximum(c, 1.0))


def _fin(k, part, cnt):
    return pl.pallas_call(
        _fin_body,
        grid=(1,),
        in_specs=[_full((NC, k, 128)), _full((NC, k, 128))],
        out_specs=_full((k, 128)),
        out_shape=jax.ShapeDtypeStruct((k, 128), F32),
    )(part, cnt)


def _t8_body(gpa, gpb, gcnt, ref128, w1a, w1b, w1r, b1, w2, b2, out):
    c = jnp.maximum(gcnt[0, :, 0:1] + gcnt[1, :, 0:1], 1.0)
    ga = (gpa[0] + gpa[1]) / c
    gb_ = (gpb[0] + gpb[1]) / c
    t = _relu(_dot(ga, w1a[...]) + _dot(gb_, w1b[...])
              + ref128[...] * w1r[...] + b1[...])
    out[...] = _dot(t, w2[...]) + b2[...]


def _t8(gpa, gpb, gcnt, ref128, w1a, w1b, w1r, b1, w2, b2):
    return pl.pallas_call(
        _t8_body,
        grid=(1,),
        in_specs=[_full((NC, N_GRAPHS, 128)), _full((NC, N_GRAPHS, 128)),
                  _full((NC, N_GRAPHS, 128)), _full((N_GRAPHS, 128)),
                  _full((128, 128)), _full((128, 128)), _full((1, 128)),
                  _full((1, 128)), _full((128, 128)), _full((1, 128))],
        out_specs=_full((N_GRAPHS, 128)),
        out_shape=jax.ShapeDtypeStruct((N_GRAPHS, 128), F32),
    )(gpa, gpb, gcnt, ref128, w1a, w1b, w1r, b1, w2, b2)


def _ranges(k):
    rp = (k // (NS * 8)) * 8
    rem = k - rp * NS
    return rp, rem


def _chunks(total):
    off = 0
    while off < total:
        sz = min(CH, total - off)
        yield off, sz
        off += sz


def _zero_rows(dst, zbuf, k, sid):
    rp, rem = _ranges(k)
    if rp:
        for off, sz in _chunks(rp):
            pltpu.sync_copy(zbuf.at[pl.ds(0, sz)], dst.at[pl.ds(sid * rp + off, sz)])
    if rem:
        @pl.when(sid == 0)
        def _():
            for off, sz in _chunks(rem):
                pltpu.sync_copy(zbuf.at[pl.ds(0, sz)], dst.at[pl.ds(rp * NS + off, sz)])


def _writeout(dst, src, stage, k, cid, sid):
    rp, rem = _ranges(k)
    if rp:
        for off, sz in _chunks(rp):
            pltpu.sync_copy(src.at[pl.ds(sid * rp + off, sz)], stage.at[pl.ds(0, sz)])
            pltpu.sync_copy(stage.at[pl.ds(0, sz)],
                            dst.at[cid, pl.ds(sid * rp + off, sz)])
    if rem:
        @pl.when(sid == 0)
        def _():
            for off, sz in _chunks(rem):
                pltpu.sync_copy(src.at[pl.ds(rp * NS + off, sz)], stage.at[pl.ds(0, sz)])
                pltpu.sync_copy(stage.at[pl.ds(0, sz)],
                                dst.at[cid, pl.ds(rp * NS + off, sz)])


def _make_segsum(k):
    mesh = plsc.VectorSubcoreMesh(core_axis_name="c", subcore_axis_name="s")

    def body(xh, ih, zh, oh, iv, bv, ts):
        cid = lax.axis_index("c")
        sid = lax.axis_index("s")
        wid = sid * NC + cid
        pltpu.sync_copy(zh, bv)
        _zero_rows(ts, bv, k, sid)
        plsc.subcore_barrier()

        def step(i, carry):
            r = (i * NW + wid) * CH
            pltpu.sync_copy(ih.at[pl.ds(r, CH)], iv)
            pltpu.sync_copy(xh.at[pl.ds(r, CH)], bv)
            pltpu.sync_copy(bv, ts.at[iv], add=True)
            return carry

        lax.fori_loop(0, NCHUNKS // NW, step, 0)
        plsc.subcore_barrier()
        _writeout(oh, ts, bv, k, cid, sid)

    return pl.kernel(
        body, mesh=mesh,
        out_type=[jax.ShapeDtypeStruct((NC, k, 128), F32)],
        scratch_types=[pltpu.VMEM((CH,), jnp.int32), pltpu.VMEM((CH, 128), F32),
                       pltpu.VMEM_SHARED((k, 128), F32)])


def _make_counts(k):
    mesh = plsc.VectorSubcoreMesh(core_axis_name="c", subcore_axis_name="s")

    def body(ih, zh, onesh, oh, iv, bv, ts):
        cid = lax.axis_index("c")
        sid = lax.axis_index("s")
        wid = sid * NC + cid
        pltpu.sync_copy(zh, bv)
        _zero_rows(ts, bv, k, sid)
        plsc.subcore_barrier()
        pltpu.sync_copy(onesh, bv)

        def step(i, carry):
            r = (i * NW + wid) * CH
            pltpu.sync_copy(ih.at[pl.ds(r, CH)], iv)
            pltpu.sync_copy(bv, ts.at[iv], add=True)
            return carry

        lax.fori_loop(0, NCHUNKS // NW, step, 0)
        plsc.subcore_barrier()
        _writeout(oh, ts, bv, k, cid, sid)

    return pl.kernel(
        body, mesh=mesh,
        out_type=[jax.ShapeDtypeStruct((NC, k, 128), F32)],
        scratch_types=[pltpu.VMEM((CH,), jnp.int32), pltpu.VMEM((CH, 128), F32),
                       pltpu.VMEM_SHARED((k, 128), F32)])


def _make_gather():
    mesh = plsc.VectorSubcoreMesh(core_axis_name="c", subcore_axis_name="s")
    out_type = [jax.ShapeDtypeStruct((N, 128), F32),
                jax.ShapeDtypeStruct((N, 128), F32)]
    scr = [pltpu.VMEM((CH,), jnp.int32), pltpu.VMEM((CH,), jnp.int32),
           pltpu.VMEM((CH, 128), F32), pltpu.VMEM((CH, 128), F32),
           pltpu.SemaphoreType.DMA, pltpu.SemaphoreType.DMA]

    def body(t1h, t2h, i1h, i2h, g1h, g2h, i1v, i2v, b1v, b2v, s1, s2):
        cid = lax.axis_index("c")
        sid = lax.axis_index("s")
        wid = sid * NC + cid

        def step(i, carry):
            r = (i * NW + wid) * CH
            pltpu.sync_copy(i1h.at[pl.ds(r, CH)], i1v)
            pltpu.sync_copy(i2h.at[pl.ds(r, CH)], i2v)
            d1 = pltpu.async_copy(t1h.at[i1v], b1v, s1)
            d2 = pltpu.async_copy(t2h.at[i2v], b2v, s2)
            d1.wait()
            d2.wait()
            pltpu.sync_copy(b1v, g1h.at[pl.ds(r, CH)])
            pltpu.sync_copy(b2v, g2h.at[pl.ds(r, CH)])
            return carry

        lax.fori_loop(0, NCHUNKS // NW, step, 0)

    return pl.kernel(body, mesh=mesh, out_type=out_type, scratch_types=scr)


def kernel(atom_feat, rdf_feat, bdf_feat, atom_idx, ele_idx, graph_idx,
           ref_feat, params):
    aidx = atom_idx.astype(jnp.int32)
    eidx = ele_idx.astype(jnp.int32)
    gidx = graph_idx.astype(jnp.int32)
    p = params

    wa = p["embed_atom"]["W"].T
    ba = p["embed_atom"]["b"][None]
    wr = p["embed_rdf"]["W"].T
    brr = p["embed_rdf"]["b"][None]
    wb = p["embed_bdf"]["W"].T
    bbb = p["embed_bdf"]["b"][None]
    h0a, h0b, xr, xb = _t1(atom_feat, rdf_feat, bdf_feat,
                           wa, ba, wr, brr, wb, bbb)

    za = jnp.zeros((CH, 128), F32)
    ones = jnp.ones((CH, 128), F32)
    gather_ae = _make_gather()
    seg_a = _make_segsum(N_ATOMS)
    seg_e = _make_segsum(N_ELE_P)
    acnt = _make_counts(N_ATOMS)(aidx, za, ones)[0]
    ecnt = _make_counts(N_ELE_P)(eidx, za, ones)[0]

    xa, xbh = h0a, h0b
    for li, mod in enumerate((p["dl1"], p["dl2"])):
        (ap,) = seg_a(xa, aidx, za)
        (ep,) = seg_e(xbh, eidx, za)
        A = _fin(N_ATOMS, ap, acnt)
        E = _fin(N_ELE_P, ep, ecnt)
        PA, PE = gather_ae(A, E, aidx, eidx)

        wtr = mod["rdf"]["fc1"]["W"].T
        wtb = mod["bdf"]["fc1"]["W"].T
        u, v, st = _t3(xr, xb, PA, PE,
                       wtr[:128], wtr[128:256], wtr[256:],
                       mod["rdf"]["fc1"]["b"][None],
                       wtb[:128], wtb[128:256], wtb[256:],
                       mod["bdf"]["fc1"]["b"][None])
        wm = mod["fc1"]["W"].T
        xa, xbh = _t4(u, v, xa, xbh, st,
                      mod["rdf"]["bn_gamma"][None], mod["rdf"]["bn_beta"][None],
                      mod["bdf"]["bn_gamma"][None], mod["bdf"]["bn_beta"][None],
                      wm[:256], wm[256:], mod["fc1"]["b"][None])

    (gpa,) = _make_segsum(N_GRAPHS)(xa, gidx, za)
    (gpb,) = _make_segsum(N_GRAPHS)(xbh, gidx, za)
    gcnt = _make_counts(N_GRAPHS)(gidx, za, ones)[0]

    w1t = p["fc1"]["W"].T
    w1a = jnp.pad(w1t[:128], ((0, 0), (0, 112)))
    w1b = jnp.pad(w1t[128:256], ((0, 0), (0, 112)))
    w1r = jnp.pad(w1t[256:257], ((0, 0), (0, 112)))
    b1 = jnp.pad(p["fc1"]["b"][None], ((0, 0), (0, 112)))
    w2 = jnp.pad(p["fc2"]["W"].T, ((0, 112), (0, 127)))
    b2 = jnp.pad(p["fc2"]["b"][None], ((0, 0), (0, 127)))
    ref128 = jnp.broadcast_to(ref_feat, (N_GRAPHS, 128))
    out = _t8(gpa, gpb, gcnt, ref128, w1a, w1b, w1r, b1, w2, b2)
    return out[:, :1]


# speedup vs baseline: 1.9036x; 1.0676x over previous
import jax
import jax.numpy as jnp
from jax import lax
from jax.experimental import pallas as pl
from jax.experimental.pallas import tpu as pltpu
from jax.experimental.pallas import tpu_sc as plsc

F32 = jnp.float32
N = 320000
AE = 128
N_ATOMS = 10000
N_ELE = 100
N_ELE_P = 104
N_GRAPHS = 1024
NC, NS = 2, 16
NW = NC * NS
CH = 128
NCHUNKS = N // CH
ITERS = -(-NCHUNKS // NW)
BR = 1280
GRID = N // BR


def _relu(x):
    return jnp.maximum(x, 0.0)


def _dot(a, b):
    return jnp.dot(a, b, preferred_element_type=F32)


def _row_spec(b, w):
    return pl.BlockSpec((b, w), lambda i: (i, 0))


def _full(shape):
    return pl.BlockSpec(shape, lambda i: (0,) * len(shape))


def _t1_body(af, rdf, bdf, wa, ba, wr, br_, wb, bb, h0a, h0b, xr, xb):
    a = af[...]
    h0a[...] = _relu(_dot(a[:, :AE], wa[...]) + ba[...])
    h0b[...] = _relu(_dot(a[:, AE:], wa[...]) + ba[...])
    xr[...] = jnp.clip(_dot(rdf[...], wr[...]) + br_[...], 0.0, 6.0)
    xb[...] = jnp.clip(_dot(bdf[...], wb[...]) + bb[...], 0.0, 6.0)


def _t1(af, rdf, bdf, wa, ba, wr, br_, wb, bb):
    return pl.pallas_call(
        _t1_body,
        grid=(GRID,),
        in_specs=[_row_spec(BR, 256), _row_spec(BR, 128), _row_spec(BR, 128),
                  _full((128, 128)), _full((1, 128)),
                  _full((128, 128)), _full((1, 128)),
                  _full((128, 128)), _full((1, 128))],
        out_specs=[_row_spec(BR, 128)] * 4,
        out_shape=[jax.ShapeDtypeStruct((N, 128), F32)] * 4,
    )(af, rdf, bdf, wa, ba, wr, br_, wb, bb)


def _t3_body(xr, xb, pa, pe, wdr, war, wer, br_, wdb, wab, web, bb,
             u_ref, v_ref, st_ref):
    i = pl.program_id(0)
    u = (_dot(xr[...], wdr[...]) + _dot(pa[...], war[...])
         + _dot(pe[...], wer[...]) + br_[...])
    v = (_dot(xb[...], wdb[...]) + _dot(pa[...], wab[...])
         + _dot(pe[...], web[...]) + bb[...])
    u_ref[...] = u
    v_ref[...] = v
    su = jnp.sum(u, axis=0, keepdims=True)
    squ = jnp.sum(u * u, axis=0, keepdims=True)
    sv = jnp.sum(v, axis=0, keepdims=True)
    sqv = jnp.sum(v * v, axis=0, keepdims=True)
    rows = jnp.concatenate([su, squ, sv, sqv, jnp.zeros((4, 2 * AE), F32)], axis=0)

    @pl.when(i == 0)
    def _():
        st_ref[...] = jnp.zeros_like(st_ref)

    st_ref[...] += rows


def _t3(xr, xb, pa, pe, wdr, war, wer, br_, wdb, wab, web, bb):
    return pl.pallas_call(
        _t3_body,
        grid=(GRID,),
        in_specs=[_row_spec(BR, 128)] * 4
        + [_full((128, 256)), _full((128, 256)), _full((128, 256)), _full((1, 256)),
           _full((128, 256)), _full((128, 256)), _full((128, 256)), _full((1, 256))],
        out_specs=[_row_spec(BR, 256), _row_spec(BR, 256), _full((8, 256))],
        out_shape=[jax.ShapeDtypeStruct((N, 256), F32),
                   jax.ShapeDtypeStruct((N, 256), F32),
                   jax.ShapeDtypeStruct((8, 256), F32)],
    )(xr, xb, pa, pe, wdr, war, wer, br_, wdb, wab, web, bb)


def _t4_body(u, v, h0a, h0b, st, gr, btr, gb, btb, wm1, wm2, bm, h1a, h1b):
    st_ = st[...]
    inv_n = 1.0 / N
    mu = st_[0:1] * inv_n
    vu = st_[1:2] * inv_n - mu * mu
    su_ = gr[...] * lax.rsqrt(vu + 1e-5)
    shu = btr[...] - mu * su_
    mv = st_[2:3] * inv_n
    vv = st_[3:4] * inv_n - mv * mv
    sv_ = gb[...] * lax.rsqrt(vv + 1e-5)
    shv = btb[...] - mv * sv_
    h0 = jnp.concatenate([h0a[...], h0b[...]], axis=1)
    x1 = _relu(u[...] * su_ + shu + h0)
    x2 = _relu(v[...] * sv_ + shv + h0)
    h1 = _relu(_dot(x1, wm1[...]) + _dot(x2, wm2[...]) + bm[...])
    h1a[...] = h1[:, :AE]
    h1b[...] = h1[:, AE:]


def _t4(u, v, h0a, h0b, st, gr, btr, gb, btb, wm1, wm2, bm):
    return pl.pallas_call(
        _t4_body,
        grid=(GRID,),
        in_specs=[_row_spec(BR, 256), _row_spec(BR, 256),
                  _row_spec(BR, 128), _row_spec(BR, 128),
                  _full((8, 256)),
                  _full((1, 256)), _full((1, 256)), _full((1, 256)), _full((1, 256)),
                  _full((256, 256)), _full((256, 256)), _full((1, 256))],
        out_specs=[_row_spec(BR, 128)] * 2,
        out_shape=[jax.ShapeDtypeStruct((N, 128), F32)] * 2,
    )(u, v, h0a, h0b, st, gr, btr, gb, btb, wm1, wm2, bm)


def _fin_body(part, cnt, out):
    s = part[0] + part[1]
    c = cnt[0, :, 0:1] + cnt[1, :, 0:1]
    out[...] = _relu(s / jnp.maximum(c, 1.0))


def _fin(k, part, cnt):
    return pl.pallas_call(
        _fin_body,
        grid=(1,),
        in_specs=[_full((NC, k, 128)), _full((NC, k, 128))],
        out_specs=_full((k, 128)),
        out_shape=jax.ShapeDtypeStruct((k, 128), F32),
    )(part, cnt)


def _t8_body(gpa, gpb, gcnt, ref128, w1a, w1b, w1r, b1, w2, b2, out):
    c = jnp.maximum(gcnt[0, :, 0:1] + gcnt[1, :, 0:1], 1.0)
    ga = (gpa[0] + gpa[1]) / c
    gb_ = (gpb[0] + gpb[1]) / c
    t = _relu(_dot(ga, w1a[...]) + _dot(gb_, w1b[...])
              + ref128[...] * w1r[...] + b1[...])
    out[...] = _dot(t, w2[...]) + b2[...]


def _t8(gpa, gpb, gcnt, ref128, w1a, w1b, w1r, b1, w2, b2):
    return pl.pallas_call(
        _t8_body,
        grid=(1,),
        in_specs=[_full((NC, N_GRAPHS, 128)), _full((NC, N_GRAPHS, 128)),
                  _full((NC, N_GRAPHS, 128)), _full((N_GRAPHS, 128)),
                  _full((128, 128)), _full((128, 128)), _full((1, 128)),
                  _full((1, 128)), _full((128, 128)), _full((1, 128))],
        out_specs=_full((N_GRAPHS, 128)),
        out_shape=jax.ShapeDtypeStruct((N_GRAPHS, 128), F32),
    )(gpa, gpb, gcnt, ref128, w1a, w1b, w1r, b1, w2, b2)


def _ranges(k):
    rp = (k // (NS * 8)) * 8
    rem = k - rp * NS
    return rp, rem


def _chunks(total):
    off = 0
    while off < total:
        sz = min(CH, total - off)
        yield off, sz
        off += sz


def _zero_rows(dst, zbuf, k, sid):
    rp, rem = _ranges(k)
    if rp:
        for off, sz in _chunks(rp):
            pltpu.sync_copy(zbuf.at[pl.ds(0, sz)], dst.at[pl.ds(sid * rp + off, sz)])
    if rem:
        @pl.when(sid == 0)
        def _():
            for off, sz in _chunks(rem):
                pltpu.sync_copy(zbuf.at[pl.ds(0, sz)], dst.at[pl.ds(rp * NS + off, sz)])


def _writeout(dst, src, stage, k, cid, sid):
    rp, rem = _ranges(k)
    if rp:
        for off, sz in _chunks(rp):
            pltpu.sync_copy(src.at[pl.ds(sid * rp + off, sz)], stage.at[pl.ds(0, sz)])
            pltpu.sync_copy(stage.at[pl.ds(0, sz)],
                            dst.at[cid, pl.ds(sid * rp + off, sz)])
    if rem:
        @pl.when(sid == 0)
        def _():
            for off, sz in _chunks(rem):
                pltpu.sync_copy(src.at[pl.ds(rp * NS + off, sz)], stage.at[pl.ds(0, sz)])
                pltpu.sync_copy(stage.at[pl.ds(0, sz)],
                                dst.at[cid, pl.ds(rp * NS + off, sz)])


def _make_segsum2(k1, k2):
    mesh = plsc.VectorSubcoreMesh(core_axis_name="c", subcore_axis_name="s")

    def body(x1h, x2h, i1h, i2h, zh, o1h, o2h, i1v, i2v, bv, ts1, ts2):
        cid = lax.axis_index("c")
        sid = lax.axis_index("s")
        wid = sid * NC + cid
        pltpu.sync_copy(zh, bv)
        _zero_rows(ts1, bv, k1, sid)
        _zero_rows(ts2, bv, k2, sid)
        plsc.subcore_barrier()

        def step(i, carry):
            c = i * NW + wid

            @pl.when(c < NCHUNKS)
            def _():
                r = c * CH
                pltpu.sync_copy(i1h.at[pl.ds(r, CH)], i1v)
                pltpu.sync_copy(i2h.at[pl.ds(r, CH)], i2v)
                pltpu.sync_copy(x1h.at[pl.ds(r, CH)], bv)
                pltpu.sync_copy(bv, ts1.at[i1v], add=True)
                pltpu.sync_copy(x2h.at[pl.ds(r, CH)], bv)
                pltpu.sync_copy(bv, ts2.at[i2v], add=True)

            return carry

        lax.fori_loop(0, ITERS, step, 0)
        plsc.subcore_barrier()
        _writeout(o1h, ts1, bv, k1, cid, sid)
        _writeout(o2h, ts2, bv, k2, cid, sid)

    return pl.kernel(
        body, mesh=mesh,
        out_type=[jax.ShapeDtypeStruct((NC, k1, 128), F32),
                  jax.ShapeDtypeStruct((NC, k2, 128), F32)],
        scratch_types=[pltpu.VMEM((CH,), jnp.int32), pltpu.VMEM((CH,), jnp.int32),
                       pltpu.VMEM((CH, 128), F32),
                       pltpu.VMEM_SHARED((k1, 128), F32),
                       pltpu.VMEM_SHARED((k2, 128), F32)])


def _make_counts3():
    mesh = plsc.VectorSubcoreMesh(core_axis_name="c", subcore_axis_name="s")
    ks = (N_ATOMS, N_ELE_P, N_GRAPHS)

    def body(iah, ieh, igh, zh, onesh, oah, oeh, ogh,
             iav, iev, igv, bv, tsa, tse, tsg):
        cid = lax.axis_index("c")
        sid = lax.axis_index("s")
        wid = sid * NC + cid
        pltpu.sync_copy(zh, bv)
        for ts, k in zip((tsa, tse, tsg), ks):
            _zero_rows(ts, bv, k, sid)
        plsc.subcore_barrier()
        pltpu.sync_copy(onesh, bv)

        def step(i, carry):
            c = i * NW + wid

            @pl.when(c < NCHUNKS)
            def _():
                r = c * CH
                pltpu.sync_copy(iah.at[pl.ds(r, CH)], iav)
                pltpu.sync_copy(ieh.at[pl.ds(r, CH)], iev)
                pltpu.sync_copy(igh.at[pl.ds(r, CH)], igv)
                pltpu.sync_copy(bv, tsa.at[iav], add=True)
                pltpu.sync_copy(bv, tse.at[iev], add=True)
                pltpu.sync_copy(bv, tsg.at[igv], add=True)

            return carry

        lax.fori_loop(0, ITERS, step, 0)
        plsc.subcore_barrier()
        for oh, ts, k in zip((oah, oeh, ogh), (tsa, tse, tsg), ks):
            _writeout(oh, ts, bv, k, cid, sid)

    return pl.kernel(
        body, mesh=mesh,
        out_type=[jax.ShapeDtypeStruct((NC, k, 128), F32) for k in ks],
        scratch_types=[pltpu.VMEM((CH,), jnp.int32), pltpu.VMEM((CH,), jnp.int32),
                       pltpu.VMEM((CH,), jnp.int32), pltpu.VMEM((CH, 128), F32),
                       pltpu.VMEM_SHARED((N_ATOMS, 128), F32),
                       pltpu.VMEM_SHARED((N_ELE_P, 128), F32),
                       pltpu.VMEM_SHARED((N_GRAPHS, 128), F32)])


def _make_gather():
    mesh = plsc.VectorSubcoreMesh(core_axis_name="c", subcore_axis_name="s")
    out_type = [jax.ShapeDtypeStruct((N, 128), F32),
                jax.ShapeDtypeStruct((N, 128), F32)]
    scr = [pltpu.VMEM((CH,), jnp.int32), pltpu.VMEM((CH,), jnp.int32),
           pltpu.VMEM((CH, 128), F32), pltpu.VMEM((CH, 128), F32),
           pltpu.SemaphoreType.DMA, pltpu.SemaphoreType.DMA]

    def body(t1h, t2h, i1h, i2h, g1h, g2h, i1v, i2v, b1v, b2v, s1, s2):
        cid = lax.axis_index("c")
        sid = lax.axis_index("s")
        wid = sid * NC + cid

        def step(i, carry):
            c = i * NW + wid

            @pl.when(c < NCHUNKS)
            def _():
                r = c * CH
                pltpu.sync_copy(i1h.at[pl.ds(r, CH)], i1v)
                pltpu.sync_copy(i2h.at[pl.ds(r, CH)], i2v)
                d1 = pltpu.async_copy(t1h.at[i1v], b1v, s1)
                d2 = pltpu.async_copy(t2h.at[i2v], b2v, s2)
                d1.wait()
                d2.wait()
                pltpu.sync_copy(b1v, g1h.at[pl.ds(r, CH)])
                pltpu.sync_copy(b2v, g2h.at[pl.ds(r, CH)])

            return carry

        lax.fori_loop(0, ITERS, step, 0)

    return pl.kernel(body, mesh=mesh, out_type=out_type, scratch_types=scr)


def kernel(atom_feat, rdf_feat, bdf_feat, atom_idx, ele_idx, graph_idx,
           ref_feat, params):
    aidx = atom_idx.astype(jnp.int32)
    eidx = ele_idx.astype(jnp.int32)
    gidx = graph_idx.astype(jnp.int32)
    p = params

    wa = p["embed_atom"]["W"].T
    ba = p["embed_atom"]["b"][None]
    wr = p["embed_rdf"]["W"].T
    brr = p["embed_rdf"]["b"][None]
    wb = p["embed_bdf"]["W"].T
    bbb = p["embed_bdf"]["b"][None]
    h0a, h0b, xr, xb = _t1(atom_feat, rdf_feat, bdf_feat,
                           wa, ba, wr, brr, wb, bbb)

    za = jnp.zeros((CH, 128), F32)
    ones = jnp.ones((CH, 128), F32)
    gather_ae = _make_gather()
    seg_ae = _make_segsum2(N_ATOMS, N_ELE_P)
    acnt, ecnt, gcnt = _make_counts3()(aidx, eidx, gidx, za, ones)

    xa, xbh = h0a, h0b
    for li, mod in enumerate((p["dl1"], p["dl2"])):
        ap, ep = seg_ae(xa, xbh, aidx, eidx, za)
        A = _fin(N_ATOMS, ap, acnt)
        E = _fin(N_ELE_P, ep, ecnt)
        PA, PE = gather_ae(A, E, aidx, eidx)

        wtr = mod["rdf"]["fc1"]["W"].T
        wtb = mod["bdf"]["fc1"]["W"].T
        u, v, st = _t3(xr, xb, PA, PE,
                       wtr[:128], wtr[128:256], wtr[256:],
                       mod["rdf"]["fc1"]["b"][None],
                       wtb[:128], wtb[128:256], wtb[256:],
                       mod["bdf"]["fc1"]["b"][None])
        wm = mod["fc1"]["W"].T
        xa, xbh = _t4(u, v, xa, xbh, st,
                      mod["rdf"]["bn_gamma"][None], mod["rdf"]["bn_beta"][None],
                      mod["bdf"]["bn_gamma"][None], mod["bdf"]["bn_beta"][None],
                      wm[:256], wm[256:], mod["fc1"]["b"][None])

    gpa, gpb = _make_segsum2(N_GRAPHS, N_GRAPHS)(xa, xbh, gidx, gidx, za)

    w1t = p["fc1"]["W"].T
    w1a = jnp.pad(w1t[:128], ((0, 0), (0, 112)))
    w1b = jnp.pad(w1t[128:256], ((0, 0), (0, 112)))
    w1r = jnp.pad(w1t[256:257], ((0, 0), (0, 112)))
    b1 = jnp.pad(p["fc1"]["b"][None], ((0, 0), (0, 112)))
    w2 = jnp.pad(p["fc2"]["W"].T, ((0, 112), (0, 127)))
    b2 = jnp.pad(p["fc2"]["b"][None], ((0, 0), (0, 127)))
    ref128 = jnp.broadcast_to(ref_feat, (N_GRAPHS, 128))
    out = _t8(gpa, gpb, gcnt, ref128, w1a, w1b, w1r, b1, w2, b2)
    return out[:, :1]


# speedup vs baseline: 2.0748x; 1.0900x over previous
import jax
import jax.numpy as jnp
from jax import lax
from jax.experimental import pallas as pl
from jax.experimental.pallas import tpu as pltpu
from jax.experimental.pallas import tpu_sc as plsc

F32 = jnp.float32
N = 320000
AE = 128
N_ATOMS = 10000
N_ELE = 100
N_ELE_P = 104
N_GRAPHS = 1024
NC, NS = 2, 16
NW = NC * NS
CH = 128
NCHUNKS = N // CH
ITERS = -(-NCHUNKS // NW)
BR = 1280
GRID = N // BR


def _relu(x):
    return jnp.maximum(x, 0.0)


def _dot(a, b):
    return jnp.dot(a, b, preferred_element_type=F32)


def _row_spec(b, w):
    return pl.BlockSpec((b, w), lambda i: (i, 0))


def _full(shape):
    return pl.BlockSpec(shape, lambda i: (0,) * len(shape))


def _t1_body(af, rdf, bdf, wa, ba, wr, br_, wb, bb, h0a, h0b, xr, xb):
    a = af[...]
    h0a[...] = _relu(_dot(a[:, :AE], wa[...]) + ba[...])
    h0b[...] = _relu(_dot(a[:, AE:], wa[...]) + ba[...])
    xr[...] = jnp.clip(_dot(rdf[...], wr[...]) + br_[...], 0.0, 6.0)
    xb[...] = jnp.clip(_dot(bdf[...], wb[...]) + bb[...], 0.0, 6.0)


def _t1(af, rdf, bdf, wa, ba, wr, br_, wb, bb):
    return pl.pallas_call(
        _t1_body,
        grid=(GRID,),
        in_specs=[_row_spec(BR, 256), _row_spec(BR, 128), _row_spec(BR, 128),
                  _full((128, 128)), _full((1, 128)),
                  _full((128, 128)), _full((1, 128)),
                  _full((128, 128)), _full((1, 128))],
        out_specs=[_row_spec(BR, 128)] * 4,
        out_shape=[jax.ShapeDtypeStruct((N, 128), F32)] * 4,
    )(af, rdf, bdf, wa, ba, wr, br_, wb, bb)


def _t3_body(xr, xb, pa, pe, wdr, war, wer, br_, wdb, wab, web, bb,
             u_ref, v_ref, st_ref):
    i = pl.program_id(0)
    u = (_dot(xr[...], wdr[...]) + _dot(pa[...], war[...])
         + _dot(pe[...], wer[...]) + br_[...])
    v = (_dot(xb[...], wdb[...]) + _dot(pa[...], wab[...])
         + _dot(pe[...], web[...]) + bb[...])
    u_ref[...] = u
    v_ref[...] = v
    su = jnp.sum(u, axis=0, keepdims=True)
    squ = jnp.sum(u * u, axis=0, keepdims=True)
    sv = jnp.sum(v, axis=0, keepdims=True)
    sqv = jnp.sum(v * v, axis=0, keepdims=True)
    rows = jnp.concatenate([su, squ, sv, sqv, jnp.zeros((4, 2 * AE), F32)], axis=0)

    @pl.when(i == 0)
    def _():
        st_ref[...] = jnp.zeros_like(st_ref)

    st_ref[...] += rows


def _t3(xr, xb, pa, pe, wdr, war, wer, br_, wdb, wab, web, bb):
    return pl.pallas_call(
        _t3_body,
        grid=(GRID,),
        in_specs=[_row_spec(BR, 128)] * 4
        + [_full((128, 256)), _full((128, 256)), _full((128, 256)), _full((1, 256)),
           _full((128, 256)), _full((128, 256)), _full((128, 256)), _full((1, 256))],
        out_specs=[_row_spec(BR, 256), _row_spec(BR, 256), _full((8, 256))],
        out_shape=[jax.ShapeDtypeStruct((N, 256), F32),
                   jax.ShapeDtypeStruct((N, 256), F32),
                   jax.ShapeDtypeStruct((8, 256), F32)],
    )(xr, xb, pa, pe, wdr, war, wer, br_, wdb, wab, web, bb)


def _t4_body(u, v, h0a, h0b, st, gr, btr, gb, btb, wm1, wm2, bm, h1a, h1b):
    st_ = st[...]
    inv_n = 1.0 / N
    mu = st_[0:1] * inv_n
    vu = st_[1:2] * inv_n - mu * mu
    su_ = gr[...] * lax.rsqrt(vu + 1e-5)
    shu = btr[...] - mu * su_
    mv = st_[2:3] * inv_n
    vv = st_[3:4] * inv_n - mv * mv
    sv_ = gb[...] * lax.rsqrt(vv + 1e-5)
    shv = btb[...] - mv * sv_
    h0 = jnp.concatenate([h0a[...], h0b[...]], axis=1)
    x1 = _relu(u[...] * su_ + shu + h0)
    x2 = _relu(v[...] * sv_ + shv + h0)
    h1 = _relu(_dot(x1, wm1[...]) + _dot(x2, wm2[...]) + bm[...])
    h1a[...] = h1[:, :AE]
    h1b[...] = h1[:, AE:]


def _t4(u, v, h0a, h0b, st, gr, btr, gb, btb, wm1, wm2, bm):
    return pl.pallas_call(
        _t4_body,
        grid=(GRID,),
        in_specs=[_row_spec(BR, 256), _row_spec(BR, 256),
                  _row_spec(BR, 128), _row_spec(BR, 128),
                  _full((8, 256)),
                  _full((1, 256)), _full((1, 256)), _full((1, 256)), _full((1, 256)),
                  _full((256, 256)), _full((256, 256)), _full((1, 256))],
        out_specs=[_row_spec(BR, 128)] * 2,
        out_shape=[jax.ShapeDtypeStruct((N, 128), F32)] * 2,
    )(u, v, h0a, h0b, st, gr, btr, gb, btb, wm1, wm2, bm)


def _fin_body(part, cnt, out):
    s = part[0] + part[1]
    c = cnt[0, :, 0:1] + cnt[1, :, 0:1]
    out[...] = _relu(s / jnp.maximum(c, 1.0))


def _fin(k, part, cnt):
    return pl.pallas_call(
        _fin_body,
        grid=(1,),
        in_specs=[_full((NC, k, 128)), _full((NC, k, 128))],
        out_specs=_full((k, 128)),
        out_shape=jax.ShapeDtypeStruct((k, 128), F32),
    )(part, cnt)


def _t8_body(gpa, gpb, gcnt, ref128, w1a, w1b, w1r, b1, w2, b2, out):
    c = jnp.maximum(gcnt[0, :, 0:1] + gcnt[1, :, 0:1], 1.0)
    ga = (gpa[0] + gpa[1]) / c
    gb_ = (gpb[0] + gpb[1]) / c
    t = _relu(_dot(ga, w1a[...]) + _dot(gb_, w1b[...])
              + ref128[...] * w1r[...] + b1[...])
    out[...] = _dot(t, w2[...]) + b2[...]


def _t8(gpa, gpb, gcnt, ref128, w1a, w1b, w1r, b1, w2, b2):
    return pl.pallas_call(
        _t8_body,
        grid=(1,),
        in_specs=[_full((NC, N_GRAPHS, 128)), _full((NC, N_GRAPHS, 128)),
                  _full((NC, N_GRAPHS, 128)), _full((N_GRAPHS, 128)),
                  _full((128, 128)), _full((128, 128)), _full((1, 128)),
                  _full((1, 128)), _full((128, 128)), _full((1, 128))],
        out_specs=_full((N_GRAPHS, 128)),
        out_shape=jax.ShapeDtypeStruct((N_GRAPHS, 128), F32),
    )(gpa, gpb, gcnt, ref128, w1a, w1b, w1r, b1, w2, b2)


def _ranges(k):
    rp = (k // (NS * 8)) * 8
    rem = k - rp * NS
    return rp, rem


def _chunks(total):
    off = 0
    while off < total:
        sz = min(CH, total - off)
        yield off, sz
        off += sz


def _zero_rows(dst, zbuf, k, sid):
    rp, rem = _ranges(k)
    if rp:
        for off, sz in _chunks(rp):
            pltpu.sync_copy(zbuf.at[pl.ds(0, sz)], dst.at[pl.ds(sid * rp + off, sz)])
    if rem:
        @pl.when(sid == 0)
        def _():
            for off, sz in _chunks(rem):
                pltpu.sync_copy(zbuf.at[pl.ds(0, sz)], dst.at[pl.ds(rp * NS + off, sz)])


def _writeout(dst, src, stage, k, cid, sid):
    rp, rem = _ranges(k)
    if rp:
        for off, sz in _chunks(rp):
            pltpu.sync_copy(src.at[pl.ds(sid * rp + off, sz)], stage.at[pl.ds(0, sz)])
            pltpu.sync_copy(stage.at[pl.ds(0, sz)],
                            dst.at[cid, pl.ds(sid * rp + off, sz)])
    if rem:
        @pl.when(sid == 0)
        def _():
            for off, sz in _chunks(rem):
                pltpu.sync_copy(src.at[pl.ds(rp * NS + off, sz)], stage.at[pl.ds(0, sz)])
                pltpu.sync_copy(stage.at[pl.ds(0, sz)],
                                dst.at[cid, pl.ds(rp * NS + off, sz)])


def _make_segsum2(k1, k2):
    mesh = plsc.VectorSubcoreMesh(core_axis_name="c", subcore_axis_name="s")

    def body(x1h, x2h, i1h, i2h, zh, o1h, o2h, i1v, i2v, b1v, b2v,
             ts1, ts2, sl):
        cid = lax.axis_index("c")
        sid = lax.axis_index("s")
        wid = sid * NC + cid
        pltpu.sync_copy(zh, b1v)
        _zero_rows(ts1, b1v, k1, sid)
        _zero_rows(ts2, b1v, k2, sid)
        plsc.subcore_barrier()

        def step(i, carry):
            c = i * NW + wid

            @pl.when(c < NCHUNKS)
            def _():
                r = c * CH
                d1 = pltpu.async_copy(i1h.at[pl.ds(r, CH)], i1v, sl)
                d2 = pltpu.async_copy(i2h.at[pl.ds(r, CH)], i2v, sl)
                d3 = pltpu.async_copy(x1h.at[pl.ds(r, CH)], b1v, sl)
                d4 = pltpu.async_copy(x2h.at[pl.ds(r, CH)], b2v, sl)
                d1.wait()
                d2.wait()
                d3.wait()
                d4.wait()
                pltpu.sync_copy(b1v, ts1.at[i1v], add=True)
                pltpu.sync_copy(b2v, ts2.at[i2v], add=True)

            return carry

        lax.fori_loop(0, ITERS, step, 0)
        plsc.subcore_barrier()
        _writeout(o1h, ts1, b1v, k1, cid, sid)
        _writeout(o2h, ts2, b1v, k2, cid, sid)

    return pl.kernel(
        body, mesh=mesh,
        out_type=[jax.ShapeDtypeStruct((NC, k1, 128), F32),
                  jax.ShapeDtypeStruct((NC, k2, 128), F32)],
        scratch_types=[pltpu.VMEM((CH,), jnp.int32), pltpu.VMEM((CH,), jnp.int32),
                       pltpu.VMEM((CH, 128), F32), pltpu.VMEM((CH, 128), F32),
                       pltpu.VMEM_SHARED((k1, 128), F32),
                       pltpu.VMEM_SHARED((k2, 128), F32),
                       pltpu.SemaphoreType.DMA])


def _make_counts3():
    mesh = plsc.VectorSubcoreMesh(core_axis_name="c", subcore_axis_name="s")
    ks = (N_ATOMS, N_ELE_P, N_GRAPHS)

    def body(iah, ieh, igh, zh, onesh, oah, oeh, ogh,
             iav, iev, igv, bv, tsa, tse, tsg, sl):
        cid = lax.axis_index("c")
        sid = lax.axis_index("s")
        wid = sid * NC + cid
        pltpu.sync_copy(zh, bv)
        for ts, k in zip((tsa, tse, tsg), ks):
            _zero_rows(ts, bv, k, sid)
        plsc.subcore_barrier()
        pltpu.sync_copy(onesh, bv)

        def step(i, carry):
            c = i * NW + wid

            @pl.when(c < NCHUNKS)
            def _():
                r = c * CH
                d1 = pltpu.async_copy(iah.at[pl.ds(r, CH)], iav, sl)
                d2 = pltpu.async_copy(ieh.at[pl.ds(r, CH)], iev, sl)
                d3 = pltpu.async_copy(igh.at[pl.ds(r, CH)], igv, sl)
                d1.wait()
                d2.wait()
                d3.wait()
                pltpu.sync_copy(bv, tsa.at[iav], add=True)
                pltpu.sync_copy(bv, tse.at[iev], add=True)
                pltpu.sync_copy(bv, tsg.at[igv], add=True)

            return carry

        lax.fori_loop(0, ITERS, step, 0)
        plsc.subcore_barrier()
        for oh, ts, k in zip((oah, oeh, ogh), (tsa, tse, tsg), ks):
            _writeout(oh, ts, bv, k, cid, sid)

    return pl.kernel(
        body, mesh=mesh,
        out_type=[jax.ShapeDtypeStruct((NC, k, 128), F32) for k in ks],
        scratch_types=[pltpu.VMEM((CH,), jnp.int32), pltpu.VMEM((CH,), jnp.int32),
                       pltpu.VMEM((CH,), jnp.int32), pltpu.VMEM((CH, 128), F32),
                       pltpu.VMEM_SHARED((N_ATOMS, 128), F32),
                       pltpu.VMEM_SHARED((N_ELE_P, 128), F32),
                       pltpu.VMEM_SHARED((N_GRAPHS, 128), F32),
                       pltpu.SemaphoreType.DMA])


def _make_gather():
    mesh = plsc.VectorSubcoreMesh(core_axis_name="c", subcore_axis_name="s")
    out_type = [jax.ShapeDtypeStruct((N, 128), F32),
                jax.ShapeDtypeStruct((N, 128), F32)]
    scr = [pltpu.VMEM((CH,), jnp.int32), pltpu.VMEM((CH,), jnp.int32),
           pltpu.VMEM((CH, 128), F32), pltpu.VMEM((CH, 128), F32),
           pltpu.SemaphoreType.DMA, pltpu.SemaphoreType.DMA,
           pltpu.SemaphoreType.DMA]

    def body(t1h, t2h, i1h, i2h, g1h, g2h, i1v, i2v, b1v, b2v, s1, s2, sl):
        cid = lax.axis_index("c")
        sid = lax.axis_index("s")
        wid = sid * NC + cid

        def step(i, carry):
            c = i * NW + wid

            @pl.when(c < NCHUNKS)
            def _():
                r = c * CH
                di1 = pltpu.async_copy(i1h.at[pl.ds(r, CH)], i1v, sl)
                di2 = pltpu.async_copy(i2h.at[pl.ds(r, CH)], i2v, sl)
                di1.wait()
                di2.wait()
                d1 = pltpu.async_copy(t1h.at[i1v], b1v, s1)
                d2 = pltpu.async_copy(t2h.at[i2v], b2v, s2)
                d1.wait()
                d2.wait()
                dw1 = pltpu.async_copy(b1v, g1h.at[pl.ds(r, CH)], s1)
                dw2 = pltpu.async_copy(b2v, g2h.at[pl.ds(r, CH)], s2)
                dw1.wait()
                dw2.wait()

            return carry

        lax.fori_loop(0, ITERS, step, 0)

    return pl.kernel(body, mesh=mesh, out_type=out_type, scratch_types=scr)


def kernel(atom_feat, rdf_feat, bdf_feat, atom_idx, ele_idx, graph_idx,
           ref_feat, params):
    aidx = atom_idx.astype(jnp.int32)
    eidx = ele_idx.astype(jnp.int32)
    gidx = graph_idx.astype(jnp.int32)
    p = params

    wa = p["embed_atom"]["W"].T
    ba = p["embed_atom"]["b"][None]
    wr = p["embed_rdf"]["W"].T
    brr = p["embed_rdf"]["b"][None]
    wb = p["embed_bdf"]["W"].T
    bbb = p["embed_bdf"]["b"][None]
    h0a, h0b, xr, xb = _t1(atom_feat, rdf_feat, bdf_feat,
                           wa, ba, wr, brr, wb, bbb)

    za = jnp.zeros((CH, 128), F32)
    ones = jnp.ones((CH, 128), F32)
    gather_ae = _make_gather()
    seg_ae = _make_segsum2(N_ATOMS, N_ELE_P)
    acnt, ecnt, gcnt = _make_counts3()(aidx, eidx, gidx, za, ones)

    xa, xbh = h0a, h0b
    for li, mod in enumerate((p["dl1"], p["dl2"])):
        ap, ep = seg_ae(xa, xbh, aidx, eidx, za)
        A = _fin(N_ATOMS, ap, acnt)
        E = _fin(N_ELE_P, ep, ecnt)
        PA, PE = gather_ae(A, E, aidx, eidx)

        wtr = mod["rdf"]["fc1"]["W"].T
        wtb = mod["bdf"]["fc1"]["W"].T
        u, v, st = _t3(xr, xb, PA, PE,
                       wtr[:128], wtr[128:256], wtr[256:],
                       mod["rdf"]["fc1"]["b"][None],
                       wtb[:128], wtb[128:256], wtb[256:],
                       mod["bdf"]["fc1"]["b"][None])
        wm = mod["fc1"]["W"].T
        xa, xbh = _t4(u, v, xa, xbh, st,
                      mod["rdf"]["bn_gamma"][None], mod["rdf"]["bn_beta"][None],
                      mod["bdf"]["bn_gamma"][None], mod["bdf"]["bn_beta"][None],
                      wm[:256], wm[256:], mod["fc1"]["b"][None])

    gpa, gpb = _make_segsum2(N_GRAPHS, N_GRAPHS)(xa, xbh, gidx, gidx, za)

    w1t = p["fc1"]["W"].T
    w1a = jnp.pad(w1t[:128], ((0, 0), (0, 112)))
    w1b = jnp.pad(w1t[128:256], ((0, 0), (0, 112)))
    w1r = jnp.pad(w1t[256:257], ((0, 0), (0, 112)))
    b1 = jnp.pad(p["fc1"]["b"][None], ((0, 0), (0, 112)))
    w2 = jnp.pad(p["fc2"]["W"].T, ((0, 112), (0, 127)))
    b2 = jnp.pad(p["fc2"]["b"][None], ((0, 0), (0, 127)))
    ref128 = jnp.broadcast_to(ref_feat, (N_GRAPHS, 128))
    out = _t8(gpa, gpb, gcnt, ref128, w1a, w1b, w1r, b1, w2, b2)
    return out[:, :1]


# speedup vs baseline: 2.0900x; 1.0073x over previous
import jax
import jax.numpy as jnp
from jax import lax
from jax.experimental import pallas as pl
from jax.experimental.pallas import tpu as pltpu
from jax.experimental.pallas import tpu_sc as plsc

F32 = jnp.float32
N = 320000
AE = 128
N_ATOMS = 10000
N_ELE = 100
N_ELE_P = 104
N_GRAPHS = 1024
NC, NS = 2, 16
NW = NC * NS
CH = 128
NCHUNKS = N // CH
ITERS = -(-NCHUNKS // NW)
BR = 1280
GRID = N // BR


def _relu(x):
    return jnp.maximum(x, 0.0)


def _dot(a, b):
    return jnp.dot(a, b, preferred_element_type=F32)


def _row_spec(b, w):
    return pl.BlockSpec((b, w), lambda i: (i, 0))


def _full(shape):
    return pl.BlockSpec(shape, lambda i: (0,) * len(shape))


def _t1_body(af, rdf, bdf, wa, ba, wr, br_, wb, bb, h0a, h0b, xr, xb):
    a = af[...]
    h0a[...] = _relu(_dot(a[:, :AE], wa[...]) + ba[...])
    h0b[...] = _relu(_dot(a[:, AE:], wa[...]) + ba[...])
    xr[...] = jnp.clip(_dot(rdf[...], wr[...]) + br_[...], 0.0, 6.0)
    xb[...] = jnp.clip(_dot(bdf[...], wb[...]) + bb[...], 0.0, 6.0)


def _t1(af, rdf, bdf, wa, ba, wr, br_, wb, bb):
    return pl.pallas_call(
        _t1_body,
        grid=(GRID,),
        in_specs=[_row_spec(BR, 256), _row_spec(BR, 128), _row_spec(BR, 128),
                  _full((128, 128)), _full((1, 128)),
                  _full((128, 128)), _full((1, 128)),
                  _full((128, 128)), _full((1, 128))],
        out_specs=[_row_spec(BR, 128)] * 4,
        out_shape=[jax.ShapeDtypeStruct((N, 128), F32)] * 4,
    )(af, rdf, bdf, wa, ba, wr, br_, wb, bb)


def _t3_body(xr, xb, pa, pe, wdr, war, wer, br_, wdb, wab, web, bb,
             u_ref, v_ref, st_ref):
    i = pl.program_id(0)
    u = (_dot(xr[...], wdr[...]) + _dot(pa[...], war[...])
         + _dot(pe[...], wer[...]) + br_[...])
    v = (_dot(xb[...], wdb[...]) + _dot(pa[...], wab[...])
         + _dot(pe[...], web[...]) + bb[...])
    u_ref[...] = u
    v_ref[...] = v
    su = jnp.sum(u, axis=0, keepdims=True)
    squ = jnp.sum(u * u, axis=0, keepdims=True)
    sv = jnp.sum(v, axis=0, keepdims=True)
    sqv = jnp.sum(v * v, axis=0, keepdims=True)
    rows = jnp.concatenate([su, squ, sv, sqv, jnp.zeros((4, 2 * AE), F32)], axis=0)

    @pl.when(i == 0)
    def _():
        st_ref[...] = jnp.zeros_like(st_ref)

    st_ref[...] += rows


def _t3(xr, xb, pa, pe, wdr, war, wer, br_, wdb, wab, web, bb):
    return pl.pallas_call(
        _t3_body,
        grid=(GRID,),
        in_specs=[_row_spec(BR, 128)] * 4
        + [_full((128, 256)), _full((128, 256)), _full((128, 256)), _full((1, 256)),
           _full((128, 256)), _full((128, 256)), _full((128, 256)), _full((1, 256))],
        out_specs=[_row_spec(BR, 256), _row_spec(BR, 256), _full((8, 256))],
        out_shape=[jax.ShapeDtypeStruct((N, 256), F32),
                   jax.ShapeDtypeStruct((N, 256), F32),
                   jax.ShapeDtypeStruct((8, 256), F32)],
    )(xr, xb, pa, pe, wdr, war, wer, br_, wdb, wab, web, bb)


def _t4_body(u, v, h0a, h0b, st, gr, btr, gb, btb, wm1, wm2, bm, h1a, h1b):
    st_ = st[...]
    inv_n = 1.0 / N
    mu = st_[0:1] * inv_n
    vu = st_[1:2] * inv_n - mu * mu
    su_ = gr[...] * lax.rsqrt(vu + 1e-5)
    shu = btr[...] - mu * su_
    mv = st_[2:3] * inv_n
    vv = st_[3:4] * inv_n - mv * mv
    sv_ = gb[...] * lax.rsqrt(vv + 1e-5)
    shv = btb[...] - mv * sv_
    h0 = jnp.concatenate([h0a[...], h0b[...]], axis=1)
    x1 = _relu(u[...] * su_ + shu + h0)
    x2 = _relu(v[...] * sv_ + shv + h0)
    h1 = _relu(_dot(x1, wm1[...]) + _dot(x2, wm2[...]) + bm[...])
    h1a[...] = h1[:, :AE]
    h1b[...] = h1[:, AE:]


def _t4(u, v, h0a, h0b, st, gr, btr, gb, btb, wm1, wm2, bm):
    return pl.pallas_call(
        _t4_body,
        grid=(GRID,),
        in_specs=[_row_spec(BR, 256), _row_spec(BR, 256),
                  _row_spec(BR, 128), _row_spec(BR, 128),
                  _full((8, 256)),
                  _full((1, 256)), _full((1, 256)), _full((1, 256)), _full((1, 256)),
                  _full((256, 256)), _full((256, 256)), _full((1, 256))],
        out_specs=[_row_spec(BR, 128)] * 2,
        out_shape=[jax.ShapeDtypeStruct((N, 128), F32)] * 2,
    )(u, v, h0a, h0b, st, gr, btr, gb, btb, wm1, wm2, bm)


def _fin_body(part, cnt, out):
    s = part[0] + part[1]
    c = cnt[0, :, 0:1] + cnt[1, :, 0:1]
    out[...] = _relu(s / jnp.maximum(c, 1.0))


def _fin(k, part, cnt):
    return pl.pallas_call(
        _fin_body,
        grid=(1,),
        in_specs=[_full((NC, k, 128)), _full((NC, k, 128))],
        out_specs=_full((k, 128)),
        out_shape=jax.ShapeDtypeStruct((k, 128), F32),
    )(part, cnt)


def _t8_body(gpa, gpb, gcnt, ref128, w1a, w1b, w1r, b1, w2, b2, out):
    c = jnp.maximum(gcnt[0, :, 0:1] + gcnt[1, :, 0:1], 1.0)
    ga = (gpa[0] + gpa[1]) / c
    gb_ = (gpb[0] + gpb[1]) / c
    t = _relu(_dot(ga, w1a[...]) + _dot(gb_, w1b[...])
              + ref128[...] * w1r[...] + b1[...])
    out[...] = _dot(t, w2[...]) + b2[...]


def _t8(gpa, gpb, gcnt, ref128, w1a, w1b, w1r, b1, w2, b2):
    return pl.pallas_call(
        _t8_body,
        grid=(1,),
        in_specs=[_full((NC, N_GRAPHS, 128)), _full((NC, N_GRAPHS, 128)),
                  _full((NC, N_GRAPHS, 128)), _full((N_GRAPHS, 128)),
                  _full((128, 128)), _full((128, 128)), _full((1, 128)),
                  _full((1, 128)), _full((128, 128)), _full((1, 128))],
        out_specs=_full((N_GRAPHS, 128)),
        out_shape=jax.ShapeDtypeStruct((N_GRAPHS, 128), F32),
    )(gpa, gpb, gcnt, ref128, w1a, w1b, w1r, b1, w2, b2)


def _ranges(k):
    rp = (k // (NS * 8)) * 8
    rem = k - rp * NS
    return rp, rem


def _chunks(total):
    off = 0
    while off < total:
        sz = min(CH, total - off)
        yield off, sz
        off += sz


def _zero_rows(dst, zbuf, k, sid):
    rp, rem = _ranges(k)
    if rp:
        for off, sz in _chunks(rp):
            pltpu.sync_copy(zbuf.at[pl.ds(0, sz)], dst.at[pl.ds(sid * rp + off, sz)])
    if rem:
        @pl.when(sid == 0)
        def _():
            for off, sz in _chunks(rem):
                pltpu.sync_copy(zbuf.at[pl.ds(0, sz)], dst.at[pl.ds(rp * NS + off, sz)])


def _writeout(dst, src, stage, k, cid, sid):
    rp, rem = _ranges(k)
    if rp:
        for off, sz in _chunks(rp):
            pltpu.sync_copy(src.at[pl.ds(sid * rp + off, sz)], stage.at[pl.ds(0, sz)])
            pltpu.sync_copy(stage.at[pl.ds(0, sz)],
                            dst.at[cid, pl.ds(sid * rp + off, sz)])
    if rem:
        @pl.when(sid == 0)
        def _():
            for off, sz in _chunks(rem):
                pltpu.sync_copy(src.at[pl.ds(rp * NS + off, sz)], stage.at[pl.ds(0, sz)])
                pltpu.sync_copy(stage.at[pl.ds(0, sz)],
                                dst.at[cid, pl.ds(rp * NS + off, sz)])


def _make_segsum2(k1, k2):
    mesh = plsc.VectorSubcoreMesh(core_axis_name="c", subcore_axis_name="s")

    def body(x1h, x2h, i1h, i2h, zh, o1h, o2h, i1v, i2v, b1v, b2v,
             ts1, ts2, sl, ss):
        cid = lax.axis_index("c")
        sid = lax.axis_index("s")
        wid = sid * NC + cid
        pltpu.sync_copy(zh, b1v)
        _zero_rows(ts1, b1v, k1, sid)
        _zero_rows(ts2, b1v, k2, sid)
        plsc.subcore_barrier()

        def step(i, carry):
            c = i * NW + wid

            @pl.when(c < NCHUNKS)
            def _():
                r = c * CH
                d1 = pltpu.async_copy(i1h.at[pl.ds(r, CH)], i1v, sl)
                d2 = pltpu.async_copy(i2h.at[pl.ds(r, CH)], i2v, sl)
                d3 = pltpu.async_copy(x1h.at[pl.ds(r, CH)], b1v, sl)
                d4 = pltpu.async_copy(x2h.at[pl.ds(r, CH)], b2v, sl)
                d1.wait()
                d2.wait()
                d3.wait()
                d4.wait()
                e1 = pltpu.async_copy(b1v, ts1.at[i1v], ss, add=True)
                e2 = pltpu.async_copy(b2v, ts2.at[i2v], ss, add=True)
                e1.wait()
                e2.wait()

            return carry

        lax.fori_loop(0, ITERS, step, 0)
        plsc.subcore_barrier()
        _writeout(o1h, ts1, b1v, k1, cid, sid)
        _writeout(o2h, ts2, b1v, k2, cid, sid)

    return pl.kernel(
        body, mesh=mesh,
        out_type=[jax.ShapeDtypeStruct((NC, k1, 128), F32),
                  jax.ShapeDtypeStruct((NC, k2, 128), F32)],
        scratch_types=[pltpu.VMEM((CH,), jnp.int32), pltpu.VMEM((CH,), jnp.int32),
                       pltpu.VMEM((CH, 128), F32), pltpu.VMEM((CH, 128), F32),
                       pltpu.VMEM_SHARED((k1, 128), F32),
                       pltpu.VMEM_SHARED((k2, 128), F32),
                       pltpu.SemaphoreType.DMA, pltpu.SemaphoreType.DMA])


def _make_counts3():
    mesh = plsc.VectorSubcoreMesh(core_axis_name="c", subcore_axis_name="s")
    ks = (N_ATOMS, N_ELE_P, N_GRAPHS)

    def body(iah, ieh, igh, zh, onesh, oah, oeh, ogh,
             iav, iev, igv, bv, tsa, tse, tsg, sl, ss):
        cid = lax.axis_index("c")
        sid = lax.axis_index("s")
        wid = sid * NC + cid
        pltpu.sync_copy(zh, bv)
        for ts, k in zip((tsa, tse, tsg), ks):
            _zero_rows(ts, bv, k, sid)
        plsc.subcore_barrier()
        pltpu.sync_copy(onesh, bv)

        def step(i, carry):
            c = i * NW + wid

            @pl.when(c < NCHUNKS)
            def _():
                r = c * CH
                d1 = pltpu.async_copy(iah.at[pl.ds(r, CH)], iav, sl)
                d2 = pltpu.async_copy(ieh.at[pl.ds(r, CH)], iev, sl)
                d3 = pltpu.async_copy(igh.at[pl.ds(r, CH)], igv, sl)
                d1.wait()
                d2.wait()
                d3.wait()
                e1 = pltpu.async_copy(bv, tsa.at[iav], ss, add=True)
                e2 = pltpu.async_copy(bv, tse.at[iev], ss, add=True)
                e3 = pltpu.async_copy(bv, tsg.at[igv], ss, add=True)
                e1.wait()
                e2.wait()
                e3.wait()

            return carry

        lax.fori_loop(0, ITERS, step, 0)
        plsc.subcore_barrier()
        for oh, ts, k in zip((oah, oeh, ogh), (tsa, tse, tsg), ks):
            _writeout(oh, ts, bv, k, cid, sid)

    return pl.kernel(
        body, mesh=mesh,
        out_type=[jax.ShapeDtypeStruct((NC, k, 128), F32) for k in ks],
        scratch_types=[pltpu.VMEM((CH,), jnp.int32), pltpu.VMEM((CH,), jnp.int32),
                       pltpu.VMEM((CH,), jnp.int32), pltpu.VMEM((CH, 128), F32),
                       pltpu.VMEM_SHARED((N_ATOMS, 128), F32),
                       pltpu.VMEM_SHARED((N_ELE_P, 128), F32),
                       pltpu.VMEM_SHARED((N_GRAPHS, 128), F32),
                       pltpu.SemaphoreType.DMA, pltpu.SemaphoreType.DMA])


def _make_gather():
    mesh = plsc.VectorSubcoreMesh(core_axis_name="c", subcore_axis_name="s")
    out_type = [jax.ShapeDtypeStruct((N, 128), F32),
                jax.ShapeDtypeStruct((N, 128), F32)]
    scr = [pltpu.VMEM((CH,), jnp.int32), pltpu.VMEM((CH,), jnp.int32),
           pltpu.VMEM((CH, 128), F32), pltpu.VMEM((CH, 128), F32),
           pltpu.SemaphoreType.DMA, pltpu.SemaphoreType.DMA,
           pltpu.SemaphoreType.DMA]

    def body(t1h, t2h, i1h, i2h, g1h, g2h, i1v, i2v, b1v, b2v, s1, s2, sl):
        cid = lax.axis_index("c")
        sid = lax.axis_index("s")
        wid = sid * NC + cid

        def step(i, carry):
            c = i * NW + wid

            @pl.when(c < NCHUNKS)
            def _():
                r = c * CH
                di1 = pltpu.async_copy(i1h.at[pl.ds(r, CH)], i1v, sl)
                di2 = pltpu.async_copy(i2h.at[pl.ds(r, CH)], i2v, sl)
                di1.wait()
                di2.wait()
                d1 = pltpu.async_copy(t1h.at[i1v], b1v, s1)
                d2 = pltpu.async_copy(t2h.at[i2v], b2v, s2)
                d1.wait()
                d2.wait()
                dw1 = pltpu.async_copy(b1v, g1h.at[pl.ds(r, CH)], s1)
                dw2 = pltpu.async_copy(b2v, g2h.at[pl.ds(r, CH)], s2)
                dw1.wait()
                dw2.wait()

            return carry

        lax.fori_loop(0, ITERS, step, 0)

    return pl.kernel(body, mesh=mesh, out_type=out_type, scratch_types=scr)


def kernel(atom_feat, rdf_feat, bdf_feat, atom_idx, ele_idx, graph_idx,
           ref_feat, params):
    aidx = atom_idx.astype(jnp.int32)
    eidx = ele_idx.astype(jnp.int32)
    gidx = graph_idx.astype(jnp.int32)
    p = params

    wa = p["embed_atom"]["W"].T
    ba = p["embed_atom"]["b"][None]
    wr = p["embed_rdf"]["W"].T
    brr = p["embed_rdf"]["b"][None]
    wb = p["embed_bdf"]["W"].T
    bbb = p["embed_bdf"]["b"][None]
    h0a, h0b, xr, xb = _t1(atom_feat, rdf_feat, bdf_feat,
                           wa, ba, wr, brr, wb, bbb)

    za = jnp.zeros((CH, 128), F32)
    ones = jnp.ones((CH, 128), F32)
    gather_ae = _make_gather()
    seg_ae = _make_segsum2(N_ATOMS, N_ELE_P)
    acnt, ecnt, gcnt = _make_counts3()(aidx, eidx, gidx, za, ones)

    xa, xbh = h0a, h0b
    for li, mod in enumerate((p["dl1"], p["dl2"])):
        ap, ep = seg_ae(xa, xbh, aidx, eidx, za)
        A = _fin(N_ATOMS, ap, acnt)
        E = _fin(N_ELE_P, ep, ecnt)
        PA, PE = gather_ae(A, E, aidx, eidx)

        wtr = mod["rdf"]["fc1"]["W"].T
        wtb = mod["bdf"]["fc1"]["W"].T
        u, v, st = _t3(xr, xb, PA, PE,
                       wtr[:128], wtr[128:256], wtr[256:],
                       mod["rdf"]["fc1"]["b"][None],
                       wtb[:128], wtb[128:256], wtb[256:],
                       mod["bdf"]["fc1"]["b"][None])
        wm = mod["fc1"]["W"].T
        xa, xbh = _t4(u, v, xa, xbh, st,
                      mod["rdf"]["bn_gamma"][None], mod["rdf"]["bn_beta"][None],
                      mod["bdf"]["bn_gamma"][None], mod["bdf"]["bn_beta"][None],
                      wm[:256], wm[256:], mod["fc1"]["b"][None])

    gpa, gpb = _make_segsum2(N_GRAPHS, N_GRAPHS)(xa, xbh, gidx, gidx, za)

    w1t = p["fc1"]["W"].T
    w1a = jnp.pad(w1t[:128], ((0, 0), (0, 112)))
    w1b = jnp.pad(w1t[128:256], ((0, 0), (0, 112)))
    w1r = jnp.pad(w1t[256:257], ((0, 0), (0, 112)))
    b1 = jnp.pad(p["fc1"]["b"][None], ((0, 0), (0, 112)))
    w2 = jnp.pad(p["fc2"]["W"].T, ((0, 112), (0, 127)))
    b2 = jnp.pad(p["fc2"]["b"][None], ((0, 0), (0, 127)))
    ref128 = jnp.broadcast_to(ref_feat, (N_GRAPHS, 128))
    out = _t8(gpa, gpb, gcnt, ref128, w1a, w1b, w1r, b1, w2, b2)
    return out[:, :1]
